# baseline (speedup 1.0000x reference)
import functools
import math

import jax
import jax.numpy as jnp
from jax import lax
from jax.experimental import pallas as pl
from jax.experimental.pallas import tpu as pltpu
from jax.experimental.pallas import tpu_sc as plsc

_DIM = 64
_SCALE = math.sqrt(float(_DIM))

_NC = 2
_NS = 16
_NW = _NC * _NS

_V = 1000000
_VT = (_V + 127) // 128

_COMPACT = pltpu.CompilerParams(
    use_tc_tiling_on_sc=True, needs_layout_passes=False)


def _mesh():
  return plsc.VectorSubcoreMesh(core_axis_name="c", subcore_axis_name="s")


@functools.lru_cache(maxsize=None)
def _build_k1():
  nbuf = 2

  @functools.partial(
      pl.kernel,
      mesh=_mesh(),
      out_type=jax.ShapeDtypeStruct((_VT * 128, 128), jnp.float32),
      scratch_types=(
          [pltpu.VMEM((_DIM, 128), jnp.float32)] * nbuf
          + [pltpu.VMEM((128, 128), jnp.float32)] * nbuf
          + [pltpu.SemaphoreType.DMA] * (2 * nbuf)
      ),
      compiler_params=_COMPACT,
  )
  def k1(tt_hbm, out_hbm, v0, v1, o0, o1, gs0, gs1, os0, os1):
    vbuf = (v0, v1)
    obuf = (o0, o1)
    gsem = (gs0, gs1)
    osem = (os0, os1)
    wid = lax.axis_index("s") * _NC + lax.axis_index("c")
    n_c = (_VT - wid + _NW - 1) // _NW

    def fire_in(ci, p):
      c = wid + ci * _NW
      pltpu.async_copy(
          tt_hbm.at[pl.ds(0, _DIM), pl.ds(c * 128, 128)], vbuf[p], gsem[p])

    def wait_in(p):
      pltpu.make_async_copy(
          tt_hbm.at[pl.ds(0, _DIM), pl.ds(0, 128)], vbuf[p], gsem[p]).wait()

    def fire_out(ci, p):
      c = wid + ci * _NW
      pltpu.async_copy(
          obuf[p], out_hbm.at[pl.ds(c * 128, 128)], osem[p])

    def wait_out(p):
      pltpu.make_async_copy(
          obuf[p], out_hbm.at[pl.ds(0, 128)], osem[p]).wait()

    fire_in(0, 0)
    iot = lax.iota(jnp.int32, 16)

    @pl.loop(0, n_c)
    def col_loop(ci):
      p = lax.rem(ci, 2)

      def with_bufs(p):
        vb, ob = vbuf[p], obuf[p]
        wait_in(p)

        @pl.when(ci + 1 < n_c)
        def _():
          fire_in(ci + 1, 1 - p)

        @plsc.parallel_loop(0, 512, unroll=8)
        def s_loop(s):
          j = lax.shift_right_logical(s, 3)
          v8 = s & 7
          val = vb[j, pl.ds(v8 * 16, 16)] * _SCALE
          plsc.store_scatter(ob, [v8 * 16 + iot, jnp.full((16,), j,
                                                          jnp.int32)], val)

        @pl.when(ci >= 2)
        def _():
          wait_out(p)

        fire_out(ci, p)

      lax.cond(p == 0, lambda: with_bufs(0), lambda: with_bufs(1))

    wait_out(0)
    wait_out(1)

  return k1


@functools.lru_cache(maxsize=None)
def _build_k2(n_t, n_b):
  assert n_b == _NW * 128
  t_tiles = n_t // 8
  assert t_tiles * 8 == n_t
  nbuf = 2

  @functools.partial(
      pl.kernel,
      mesh=_mesh(),
      out_type=jax.ShapeDtypeStruct((n_t, _DIM, n_b), jnp.float32),
      scratch_types=(
          [pltpu.VMEM((8, 128), jnp.int32)]
          + [pltpu.VMEM((128, 128), jnp.float32)] * nbuf
          + [pltpu.VMEM((_DIM, 128), jnp.float32)] * nbuf
          + [pltpu.SemaphoreType.DMA] * (2 * nbuf)
      ),
      compiler_params=_COMPACT,
  )
  def k2(it_hbm, t2_hbm, out_hbm, ibuf, g0, g1, o0, o1, gs0, gs1, os0, os1):
    gbuf = (g0, g1)
    obuf = (o0, o1)
    gsem = (gs0, gs1)
    osem = (os0, os1)
    wid = lax.axis_index("s") * _NC + lax.axis_index("c")
    b0 = wid * 128
    iot = lax.iota(jnp.int32, 16)

    def load_idx_tile(tt):
      pltpu.sync_copy(
          it_hbm.at[pl.ds(tt * 8, 8), pl.ds(b0, 128)], ibuf)

    def fire_gather(tr, p):
      pltpu.async_copy(t2_hbm.at[ibuf.at[tr]], gbuf[p], gsem[p])

    def wait_gather(p):
      pltpu.make_async_copy(
          t2_hbm.at[pl.ds(0, 128)], gbuf[p], gsem[p]).wait()

    def fire_out(t, p):
      pltpu.async_copy(
          obuf[p], out_hbm.at[t, pl.ds(0, _DIM), pl.ds(b0, 128)], osem[p])

    def wait_out(p):
      pltpu.make_async_copy(
          obuf[p], out_hbm.at[0, pl.ds(0, _DIM), pl.ds(0, 128)],
          osem[p]).wait()

    def compute(p):
      gb, ob = gbuf[p], obuf[p]

      @plsc.parallel_loop(0, 512, unroll=8)
      def s_loop(s):
        k = lax.shift_right_logical(s, 2)
        c = s & 3
        val = gb[k, pl.ds(c * 16, 16)]
        plsc.store_scatter(
            ob, [c * 16 + iot, jnp.full((16,), k, jnp.int32)], val)

    @pl.loop(0, t_tiles)
    def tile_loop(tt):
      load_idx_tile(tt)
      fire_gather(0, 0)
      for tr in range(8):
        p = tr % 2
        wait_gather(p)
        if tr < 7:
          fire_gather(tr + 1, 1 - p)
        compute(p)

        @pl.when((tt * 8 + tr) >= 2)
        def _():
          wait_out(p)

        fire_out(tt * 8 + tr, p)

    wait_out(0)
    wait_out(1)

  return k2


def kernel(inputs, table):
  b, t = inputs.shape
  tt = jnp.transpose(table)
  t2 = _build_k1()(tt)
  it = jnp.transpose(inputs).astype(jnp.int32)
  o3 = _build_k2(t, b)(it, t2)
  return jnp.transpose(o3, (2, 0, 1))


# speedup vs baseline: 1.6705x; 1.6705x over previous
import functools
import math

import jax
import jax.numpy as jnp
from jax import lax
from jax.experimental import pallas as pl
from jax.experimental.pallas import tpu as pltpu
from jax.experimental.pallas import tpu_sc as plsc

_DIM = 64
_SCALE = math.sqrt(float(_DIM))

_NC = 2
_NS = 16
_NW = _NC * _NS

_IDX_ROW = 128
_CHUNK_ROWS = 2
_CHUNK = _IDX_ROW * _CHUNK_ROWS
_NBUF = 4


@functools.lru_cache(maxsize=None)
def _build(n_idx_rows):
  rows_per_w = n_idx_rows // _NW
  chunks = rows_per_w // _CHUNK_ROWS
  outer = chunks // _NBUF
  assert rows_per_w * _NW == n_idx_rows
  assert chunks * _CHUNK_ROWS == rows_per_w
  assert outer * _NBUF == chunks and outer >= 3
  b_total = n_idx_rows * _IDX_ROW
  mesh = plsc.VectorSubcoreMesh(core_axis_name="c", subcore_axis_name="s")

  @functools.partial(
      pl.kernel,
      mesh=mesh,
      out_type=jax.ShapeDtypeStruct((b_total, _DIM), jnp.float32),
      scratch_types=(
          [pltpu.VMEM((_CHUNK_ROWS, _IDX_ROW), jnp.int32)] * _NBUF
          + [pltpu.VMEM((_CHUNK, _DIM), jnp.float32)] * _NBUF
          + [pltpu.SemaphoreType.DMA] * (3 * _NBUF)
      ),
      compiler_params=pltpu.CompilerParams(use_tc_tiling_on_sc=False),
  )
  def k(idx_hbm, table_hbm, out_hbm, *bufs):
    ibuf = bufs[:_NBUF]
    rbuf = bufs[_NBUF:2 * _NBUF]
    gsem = bufs[2 * _NBUF:3 * _NBUF]
    osem = bufs[3 * _NBUF:4 * _NBUF]
    isem = bufs[4 * _NBUF:5 * _NBUF]

    wid = lax.axis_index("s") * _NC + lax.axis_index("c")
    row_base = wid * rows_per_w

    def idx_row(g):
      return row_base + g * _CHUNK_ROWS

    def fire_gathers(g, b):
      del g
      for j in range(_CHUNK_ROWS):
        pltpu.async_copy(
            table_hbm.at[ibuf[b].at[j]],
            rbuf[b].at[pl.ds(j * _IDX_ROW, _IDX_ROW)],
            gsem[b],
        )

    def wait_gathers(b):
      for j in range(_CHUNK_ROWS):
        pltpu.make_async_copy(
            table_hbm.at[pl.ds(0, _IDX_ROW)],
            rbuf[b].at[pl.ds(j * _IDX_ROW, _IDX_ROW)],
            gsem[b],
        ).wait()

    def fire_idx(g, b):
      pltpu.async_copy(
          idx_hbm.at[pl.ds(idx_row(g), _CHUNK_ROWS)], ibuf[b], isem[b])

    def wait_idx(b):
      pltpu.make_async_copy(
          idx_hbm.at[pl.ds(0, _CHUNK_ROWS)], ibuf[b], isem[b]).wait()

    def fire_out(g, b):
      pltpu.async_copy(
          rbuf[b], out_hbm.at[pl.ds(idx_row(g) * _IDX_ROW, _CHUNK)], osem[b])

    def wait_out(b):
      pltpu.make_async_copy(
          rbuf[b], out_hbm.at[pl.ds(0, _CHUNK)], osem[b]).wait()

    def scale(b):
      ref = rbuf[b]

      @plsc.parallel_loop(0, _CHUNK, unroll=8)
      def _(r):
        for c in range(_DIM // 16):
          sl = pl.ds(c * 16, 16)
          ref[r, sl] = ref[r, sl] * _SCALE

    def body(g, b, first=False, fire_i=True, fire_g=True):
      wait_gathers(b)
      if fire_i:
        fire_idx(g + _NBUF, b)
      scale(b)
      fire_out(g, b)
      if fire_g:
        bn = (b + _NBUF - 1) % _NBUF
        if not first:
          wait_out(bn)
        wait_idx(bn)
        fire_gathers(g + _NBUF - 1, bn)

    for b in range(_NBUF - 1):
      pltpu.sync_copy(idx_hbm.at[pl.ds(idx_row(b), _CHUNK_ROWS)], ibuf[b])
      fire_gathers(b, b)
    fire_idx(_NBUF - 1, _NBUF - 1)

    body(0, 0, first=True)
    for b in range(1, _NBUF):
      body(b, b)

    @pl.loop(1, outer - 1)
    def _(i):
      g0 = i * _NBUF
      for b in range(_NBUF):
        body(g0 + b, b)

    gl = (outer - 1) * _NBUF
    body(gl, 0, fire_i=False)
    for b in range(1, _NBUF):
      body(gl + b, b, fire_i=False, fire_g=False)

    for b in range(_NBUF):
      wait_out(b)

  return k


def kernel(inputs, table):
  b, t = inputs.shape
  n = b * t
  idx2d = inputs.reshape(n // _IDX_ROW, _IDX_ROW).astype(jnp.int32)
  out = _build(n // _IDX_ROW)(idx2d, table)
  return out.reshape(b, t, _DIM)
